# baseline (speedup 1.0000x reference)
import jax
import jax.numpy as jnp
from jax.experimental import pallas as pl
from jax.experimental.pallas import tpu as pltpu

_B, _P, _C = 32, 8732, 21
_NCH = 4
_CB = _B // _NCH
_EPS = 1e-07


def _accumulate(lam_ref, conf_ref, temp_ref, interp_ref, loc_ref, loct_ref,
                loci_ref, acc_ref):
    lam = lam_ref[0, 0]
    conf = conf_ref[0]
    temp = temp_ref[0]
    interp = interp_ref[0] + _EPS

    left = (jnp.max(conf[1:, :], axis=0, keepdims=True)
            > conf[0:1, :]).astype(jnp.float32)
    right = (jnp.max(temp[1:, :], axis=0, keepdims=True)
             > temp[0:1, :]).astype(jnp.float32)
    inter = left * right
    only_l = left * (1.0 - right)
    only_r = right * (1.0 - left)

    mixed = lam * conf + (1.0 - lam) * temp + _EPS
    conf_eps = conf + _EPS
    temp_eps = temp + _EPS
    log_mixed = jnp.log(mixed)
    log_interp = jnp.log(interp)
    log_conf = jnp.log(conf_eps)
    log_temp = jnp.log(temp_eps)

    kl_ab = jnp.sum((interp - mixed) * (log_interp - log_mixed),
                    axis=0, keepdims=True)
    kl_l = jnp.sum(conf_eps * (log_conf - log_interp), axis=0, keepdims=True)
    kl_r = jnp.sum(temp_eps * (log_temp - log_interp), axis=0, keepdims=True)

    se_l = jnp.sum((loci_ref[0] - loc_ref[0]) ** 2, axis=0, keepdims=True)
    se_r = jnp.sum((loci_ref[0] - loct_ref[0]) ** 2, axis=0, keepdims=True)

    acc_ref[0:1, :] += kl_ab * inter
    acc_ref[1:2, :] += inter
    acc_ref[2:3, :] += (kl_l + 0.25 * se_l) * only_l
    acc_ref[3:4, :] += only_l
    acc_ref[4:5, :] += (kl_r + 0.25 * se_r) * only_r
    acc_ref[5:6, :] += only_r


def _chunk_body(lam_ref, acc_in_ref, conf_ref, temp_ref, interp_ref, loc_ref,
                loct_ref, loci_ref, acc_out_ref, scr_ref):
    b = pl.program_id(0)

    @pl.when(b == 0)
    def _init():
        scr_ref[...] = acc_in_ref[...]

    _accumulate(lam_ref, conf_ref, temp_ref, interp_ref, loc_ref, loct_ref,
                loci_ref, scr_ref)

    @pl.when(b == _CB - 1)
    def _flush():
        acc_out_ref[...] = scr_ref[...]


def _last_body(lam_ref, acc_in_ref, conf_ref, temp_ref, interp_ref, loc_ref,
               loct_ref, loci_ref, out_i_ref, out_f_ref, scr_ref):
    b = pl.program_id(0)

    @pl.when(b == 0)
    def _init():
        scr_ref[...] = acc_in_ref[...]

    _accumulate(lam_ref, conf_ref, temp_ref, interp_ref, loc_ref, loct_ref,
                loci_ref, scr_ref)

    @pl.when(b == _CB - 1)
    def _finish():
        s_ab = jnp.sum(scr_ref[0:1, :])
        cnt_i = jnp.sum(scr_ref[1:2, :])
        s_l = jnp.sum(scr_ref[2:3, :])
        cnt_l = jnp.sum(scr_ref[3:4, :])
        s_r = jnp.sum(scr_ref[4:5, :])
        cnt_r = jnp.sum(scr_ref[5:6, :])
        interp_loss = jnp.where(cnt_i > 0.0,
                                s_ab / (2.0 * jnp.maximum(cnt_i, 1.0)), 0.0)
        fix_loss = (jnp.where(cnt_l > 0.0, s_l / jnp.maximum(cnt_l, 1.0), 0.0)
                    + jnp.where(cnt_r > 0.0, s_r / jnp.maximum(cnt_r, 1.0), 0.0))
        out_i_ref[0, 0] = interp_loss
        out_f_ref[0, 0] = fix_loss


def _data_specs():
    conf_sp = pl.BlockSpec((1, _C, _P), lambda b: (b, 0, 0))
    loc_sp = pl.BlockSpec((1, 4, _P), lambda b: (b, 0, 0))
    return [conf_sp, conf_sp, conf_sp, loc_sp, loc_sp, loc_sp]


def _chunk_call(lam2d, acc, data, last):
    in_specs = ([pl.BlockSpec(memory_space=pltpu.SMEM),
                 pl.BlockSpec((8, _P), lambda b: (0, 0))] + _data_specs())
    if last:
        return pl.pallas_call(
            _last_body,
            grid=(_CB,),
            in_specs=in_specs,
            out_specs=[pl.BlockSpec(memory_space=pltpu.SMEM),
                       pl.BlockSpec(memory_space=pltpu.SMEM)],
            out_shape=[jax.ShapeDtypeStruct((1, 1), jnp.float32),
                       jax.ShapeDtypeStruct((1, 1), jnp.float32)],
            scratch_shapes=[pltpu.VMEM((8, _P), jnp.float32)],
        )(lam2d, acc, *data)
    return pl.pallas_call(
        _chunk_body,
        grid=(_CB,),
        in_specs=in_specs,
        out_specs=pl.BlockSpec((8, _P), lambda b: (0, 0)),
        out_shape=jax.ShapeDtypeStruct((8, _P), jnp.float32),
        scratch_shapes=[pltpu.VMEM((8, _P), jnp.float32)],
    )(lam2d, acc, *data)


def kernel(lam, conf, conf_flip, loc, loc_flip, conf_shuffle,
           conf_interpolation, loc_shuffle, loc_interpolation):
    del conf_flip, loc_flip
    lam2d = jnp.reshape(lam.astype(jnp.float32), (1, 1))

    def chunk(x, i):
        return jnp.swapaxes(x[i * _CB:(i + 1) * _CB], 1, 2)

    acc = jnp.zeros((8, _P), jnp.float32)
    for i in range(_NCH):
        j = (i + _NCH // 2) % _NCH
        data = [chunk(conf, i), chunk(conf_shuffle, j),
                chunk(conf_interpolation, i), chunk(loc, i),
                chunk(loc_shuffle, j), chunk(loc_interpolation, i)]
        if i < _NCH - 1:
            acc = _chunk_call(lam2d, acc, data, last=False)
        else:
            out_i, out_f = _chunk_call(lam2d, acc, data, last=True)
    return out_i.reshape(()), out_f.reshape(())
